# baseline (speedup 1.0000x reference)
import functools

import jax
import jax.numpy as jnp
from jax import lax
from jax.experimental import pallas as pl
from jax.experimental.pallas import tpu as pltpu
from jax.experimental.pallas import tpu_sc as plsc

_K = 32
_TEMP = 0.1
_INV_SQRT_D = 0.125

_L = 16
_NC = 2
_NS = 16
_N = 100000
_CHUNK = 20000
_CPR = _N // _CHUNK
_NBUF = 4
_ROWS_PER_W = 16
_TOT = _ROWS_PER_W * _CPR
_G = 5
_GRP = 25
_NVEC = _CHUNK // _L
_NGRP = _NVEC // _GRP
_NEG_INF = float("-inf")


def _iota():
    return lax.iota(jnp.int32, _L)


def _perm(x, j):
    return jnp.take_along_axis(x, _iota() ^ j, axis=0)


def _bfly(x, op):
    for j in (1, 2, 4, 8):
        x = op(x, _perm(x, j))
    return x


def _sortnet16(x):
    i = _iota()
    k = 2
    lk = 1
    while k <= _L:
        j = k >> 1
        lj = lk - 1
        while j > 0:
            p = _perm(x, j)
            take_min = ((i >> lk) & 1) == ((i >> lj) & 1)
            x = jnp.where(take_min, jnp.minimum(x, p), jnp.maximum(x, p))
            j >>= 1
            lj -= 1
        k <<= 1
        lk += 1
    return x


def _clean_asc(x):
    i = _iota()
    for j in (8, 4, 2, 1):
        p = _perm(x, j)
        low = (i & j) == 0
        x = jnp.where(low, jnp.minimum(x, p), jnp.maximum(x, p))
    return x


def _rev(x):
    return lax.rev(x, (0,))


def _merge_top32(t0, t1, s):
    sd = _rev(_sortnet16(s))
    hi = jnp.maximum(t0, sd)
    lo = jnp.minimum(t0, sd)
    t0n = _clean_asc(hi)
    lod = _rev(_clean_asc(lo))
    t1n = _clean_asc(jnp.maximum(t1, lod))
    return t0n, t1n


def _tree_max(vs):
    while len(vs) > 1:
        nxt = [jnp.maximum(vs[i], vs[i + 1])
               for i in range(0, len(vs) - 1, 2)]
        if len(vs) % 2:
            nxt.append(vs[-1])
        vs = nxt
    return vs[0]


@functools.partial(
    pl.kernel,
    out_type=jax.ShapeDtypeStruct((512 * _K,), jnp.float32),
    mesh=plsc.VectorSubcoreMesh(
        core_axis_name="c", subcore_axis_name="s",
        num_cores=_NC, num_subcores=_NS,
    ),
    scratch_types=[
        pltpu.VMEM((_L,), jnp.float32),
        pltpu.VMEM((_CHUNK,), jnp.float32),
        pltpu.VMEM((_CHUNK,), jnp.float32),
        pltpu.VMEM((_CHUNK,), jnp.float32),
        pltpu.VMEM((_CHUNK,), jnp.float32),
        pltpu.VMEM((2 * _L,), jnp.float32),
        pltpu.VMEM((2 * _L,), jnp.float32),
        pltpu.VMEM((2 * _L,), jnp.float32),
        pltpu.SemaphoreType.DMA,
        pltpu.SemaphoreType.DMA,
        pltpu.SemaphoreType.DMA,
        pltpu.SemaphoreType.DMA,
        pltpu.SemaphoreType.DMA,
    ],
)
def _sc_topk(keys_hbm, q_hbm, out_hbm, qv, bufa, bufb, bufc, bufd,
             kfirst, outr, tstate, sema, semb, semc, semd, semk):
    bufs = (bufa, bufb, bufc, bufd)
    sems = (sema, semb, semc, semd)
    wid = lax.axis_index("s") * _NC + lax.axis_index("c")
    row_base = wid * _ROWS_PER_W

    pltpu.sync_copy(q_hbm.at[pl.ds(row_base * 1, _L)], qv)

    def chunk_ref(g):
        row = row_base + g // _CPR
        c = g % _CPR
        return keys_hbm.at[pl.ds(row * _N + c * _CHUNK, _CHUNK)]

    def kfirst_ref(row):
        return keys_hbm.at[pl.ds(row * _N, 2 * _L)]

    for p in range(_NBUF):
        pltpu.async_copy(chunk_ref(jnp.int32(p)), bufs[p], sems[p])
    pltpu.async_copy(kfirst_ref(jnp.int32(row_base)), kfirst, semk)

    neg = jnp.full((_L,), _NEG_INF, jnp.float32)

    def row_scale(r):
        qreg = qv[pl.ds(0, _L)]
        s = lax.gather(
            qreg, jnp.full((_L, 1), r, jnp.int32),
            lax.GatherDimensionNumbers(
                offset_dims=(), collapsed_slice_dims=(0,),
                start_index_map=(0,)),
            slice_sizes=(1,),
            mode=lax.GatherScatterMode.PROMISE_IN_BOUNDS)
        return s * _INV_SQRT_D

    def scan_chunk(buf, tau, scale):
        def body(g, tau):
            base = g * (_L * _GRP)
            ss = [buf[pl.ds(base + i * _L, _L)] * scale
                  for i in range(_GRP)]
            gmax = _bfly(_tree_max(ss), jnp.maximum)[0]

            def on_hit(tau):
                def sub(k, tau):
                    sbase = base + k * (_L * _G)
                    s5 = [buf[pl.ds(sbase + i * _L, _L)] * scale
                          for i in range(_G)]
                    smax = _bfly(_tree_max(s5), jnp.maximum)[0]

                    def sub_hit(t):
                        t0 = tstate[pl.ds(0, _L)]
                        t1 = tstate[pl.ds(_L, _L)]
                        for s in s5:
                            t0, t1 = _merge_top32(t0, t1, s)
                        tstate[pl.ds(0, _L)] = t0
                        tstate[pl.ds(_L, _L)] = t1
                        return jnp.maximum(t, t1[0])

                    return lax.cond(smax > tau, sub_hit, lambda t: t, tau)

                return lax.fori_loop(0, _GRP // _G, sub, tau)

            return lax.cond(gmax > tau, on_hit, lambda t: t, tau)

        return lax.fori_loop(0, _NGRP, body, tau, unroll=2)

    def epilogue(r, scale):
        t0 = tstate[pl.ds(0, _L)]
        t1 = tstate[pl.ds(_L, _L)]
        d0 = _rev(t0)
        d1 = _rev(t1)
        m = t0[_L - 1]
        inv_t = jnp.float32(1.0 / _TEMP)
        e0 = jnp.exp((d0 - m) * inv_t)
        e1 = jnp.exp((d1 - m) * inv_t)
        z = _bfly(e0 + e1, jnp.add)[0]
        g0 = d0 / scale
        g1 = d1 / scale
        res0 = e0 / z * g0
        res1 = e1 / z * g1

        pltpu.make_async_copy(
            keys_hbm.at[pl.ds(0, 2 * _L)], kfirst, semk).wait()

        def qzero():
            outr[pl.ds(0, _L)] = kfirst[pl.ds(0, _L)] * (1.0 / 32.0)
            outr[pl.ds(_L, _L)] = kfirst[pl.ds(_L, _L)] * (1.0 / 32.0)

        def qnonzero():
            outr[pl.ds(0, _L)] = res0
            outr[pl.ds(_L, _L)] = res1

        outr[pl.ds(0, _L)] = scale
        qs = outr[pl.ds(0, _L)][0]
        lax.cond(qs == 0.0, qzero, qnonzero)

        @pl.when(r + 1 < _ROWS_PER_W)
        def _():
            pltpu.async_copy(kfirst_ref(row_base + r + 1), kfirst, semk)

        row = row_base + r
        pltpu.sync_copy(outr, out_hbm.at[pl.ds(row * _K, _K)])

    def pack(gg, tau):
        for p in range(_NBUF):
            g = gg * _NBUF + p
            r = g // _CPR
            c = g % _CPR
            scale = row_scale(r)

            @pl.when(c == 0)
            def _():
                tstate[pl.ds(0, _L)] = neg
                tstate[pl.ds(_L, _L)] = neg

            tau = jnp.where(c == 0, jnp.float32(_NEG_INF), tau)

            pltpu.make_async_copy(
                keys_hbm.at[pl.ds(0, _CHUNK)], bufs[p], sems[p]).wait()
            tau = scan_chunk(bufs[p], tau, scale)

            @pl.when(g + _NBUF < _TOT)
            def _():
                pltpu.async_copy(chunk_ref(g + _NBUF), bufs[p], sems[p])

            @pl.when(c == _CPR - 1)
            def _():
                epilogue(r, scale)

        return tau

    lax.fori_loop(0, _TOT // _NBUF, pack, jnp.float32(_NEG_INF))


def kernel(query, keys):
    b, d = query.shape
    n = keys.shape[-1]
    kf = keys.reshape(b * d * n)
    qf = query.reshape(b * d)
    out_flat = _sc_topk(kf, qf)
    return out_flat.reshape(b, d, _K).transpose(0, 2, 1)


# speedup vs baseline: 1.3673x; 1.3673x over previous
import functools

import jax
import jax.numpy as jnp
from jax import lax
from jax.experimental import pallas as pl
from jax.experimental.pallas import tpu as pltpu
from jax.experimental.pallas import tpu_sc as plsc

_K = 32
_TEMP = 0.1
_INV_SQRT_D = 0.125

_L = 16
_NC = 2
_NS = 16
_N = 100000
_NMAIN = 99968
_CHUNK = 4096
_NFULL = _NMAIN // _CHUNK
_PART = _NMAIN - _NFULL * _CHUNK
_NBUF = 3
_ROWS_PER_W = 16
_GRP = 32
_SUB = 8
_NEG_INF = float("-inf")


def _iota():
    return lax.iota(jnp.int32, _L)


def _perm(x, j):
    return jnp.take_along_axis(x, _iota() ^ j, axis=0)


def _bfly(x, op):
    for j in (1, 2, 4, 8):
        x = op(x, _perm(x, j))
    return x


def _sortnet16(x):
    i = _iota()
    k = 2
    lk = 1
    while k <= _L:
        j = k >> 1
        lj = lk - 1
        while j > 0:
            p = _perm(x, j)
            take_min = ((i >> lk) & 1) == ((i >> lj) & 1)
            x = jnp.where(take_min, jnp.minimum(x, p), jnp.maximum(x, p))
            j >>= 1
            lj -= 1
        k <<= 1
        lk += 1
    return x


def _clean_asc(x):
    i = _iota()
    for j in (8, 4, 2, 1):
        p = _perm(x, j)
        low = (i & j) == 0
        x = jnp.where(low, jnp.minimum(x, p), jnp.maximum(x, p))
    return x


def _rev(x):
    return lax.rev(x, (0,))


def _merge_top32(t0, t1, s):
    sd = _rev(_sortnet16(s))
    hi = jnp.maximum(t0, sd)
    lo = jnp.minimum(t0, sd)
    t0n = _clean_asc(hi)
    lod = _rev(_clean_asc(lo))
    t1n = _clean_asc(jnp.maximum(t1, lod))
    return t0n, t1n


def _tree_max(vs):
    while len(vs) > 1:
        nxt = [jnp.maximum(vs[i], vs[i + 1])
               for i in range(0, len(vs) - 1, 2)]
        if len(vs) % 2:
            nxt.append(vs[-1])
        vs = nxt
    return vs[0]


@functools.partial(
    pl.kernel,
    out_type=jax.ShapeDtypeStruct((512 * _K,), jnp.float32),
    mesh=plsc.VectorSubcoreMesh(
        core_axis_name="c", subcore_axis_name="s",
        num_cores=_NC, num_subcores=_NS,
    ),
    scratch_types=[
        pltpu.VMEM((_L,), jnp.float32),
        pltpu.VMEM((8, _CHUNK), jnp.float32),
        pltpu.VMEM((8, _CHUNK), jnp.float32),
        pltpu.VMEM((8, _CHUNK), jnp.float32),
        pltpu.VMEM((8, _PART), jnp.float32),
        pltpu.VMEM((_ROWS_PER_W * 2 * _L,), jnp.float32),
        pltpu.VMEM((_ROWS_PER_W * 2 * _L,), jnp.float32),
        pltpu.VMEM((8 * 2 * _L,), jnp.float32),
        pltpu.VMEM((8 * _L,), jnp.float32),
        pltpu.VMEM((2 * _L,), jnp.float32),
        pltpu.SemaphoreType.DMA,
        pltpu.SemaphoreType.DMA,
        pltpu.SemaphoreType.DMA,
        pltpu.SemaphoreType.DMA,
    ],
)
def _sc_topk(keys_hbm, q_hbm, head_hbm, tail_hbm, out_hbm,
             qv, bufa, bufb, bufc, bufp, hv, tv, tstate, taus, outr,
             sema, semb, semc, semp):
    bufs = (bufa, bufb, bufc)
    sems = (sema, semb, semc)
    wid = lax.axis_index("s") * _NC + lax.axis_index("c")
    row_base = wid * _ROWS_PER_W

    pltpu.sync_copy(q_hbm.at[pl.ds(row_base * 1, _L)], qv)
    pltpu.sync_copy(head_hbm.at[pl.ds(row_base * _K, _ROWS_PER_W * _K)], hv)
    pltpu.sync_copy(tail_hbm.at[pl.ds(row_base * _K, _ROWS_PER_W * _K)], tv)

    neg = jnp.full((_L,), _NEG_INF, jnp.float32)

    def row_scale(r):
        qreg = qv[pl.ds(0, _L)]
        s = lax.gather(
            qreg, jnp.full((_L, 1), r, jnp.int32),
            lax.GatherDimensionNumbers(
                offset_dims=(), collapsed_slice_dims=(0,),
                start_index_map=(0,)),
            slice_sizes=(1,),
            mode=lax.GatherScatterMode.PROMISE_IN_BOUNDS)
        return s * _INV_SQRT_D

    def scan_rows(buf, nvec, grp, block):
        ngrp = nvec // grp
        nsub = grp // _SUB

        def row(i, _):
            scale = row_scale(block * 8 + i)
            tau0 = taus[pl.ds(i * _L, _L)][0]

            def grp_body(g, tau):
                base = g * (_L * grp)
                ss = [buf[i, pl.ds(base + j * _L, _L)] * scale
                      for j in range(grp)]
                gmax = _bfly(_tree_max(ss), jnp.maximum)[0]

                def on_hit(tau):
                    def sub(k, tau):
                        sbase = base + k * (_L * _SUB)
                        s8 = [buf[i, pl.ds(sbase + j * _L, _L)] * scale
                              for j in range(_SUB)]
                        smax = _bfly(_tree_max(s8), jnp.maximum)[0]

                        def sub_hit(t):
                            t0 = tstate[pl.ds(i * 2 * _L, _L)]
                            t1 = tstate[pl.ds(i * 2 * _L + _L, _L)]
                            for s in s8:
                                t0, t1 = _merge_top32(t0, t1, s)
                            tstate[pl.ds(i * 2 * _L, _L)] = t0
                            tstate[pl.ds(i * 2 * _L + _L, _L)] = t1
                            return jnp.maximum(t, t1[0])

                        return lax.cond(smax > tau, sub_hit,
                                        lambda t: t, tau)

                    return lax.fori_loop(0, nsub, sub, tau)

                return lax.cond(gmax > tau, on_hit, lambda t: t, tau)

            tau_f = lax.fori_loop(0, ngrp, grp_body, tau0)
            taus[pl.ds(i * _L, _L)] = jnp.full((_L,), tau_f, jnp.float32)
            return 0

        lax.fori_loop(0, 8, row, 0)

    for block in (0, 1):
        rb = row_base + block * 8

        def init(j, _):
            tstate[pl.ds(j * _L, _L)] = neg
            return 0

        lax.fori_loop(0, 16, init, 0)

        def init_tau(j, _):
            taus[pl.ds(j * _L, _L)] = neg
            return 0

        lax.fori_loop(0, 8, init_tau, 0)

        for p in range(_NBUF):
            pltpu.async_copy(
                keys_hbm.at[pl.ds(rb, 8), pl.ds(p * _CHUNK, _CHUNK)],
                bufs[p], sems[p])
        pltpu.async_copy(
            keys_hbm.at[pl.ds(rb, 8), pl.ds(_NFULL * _CHUNK, _PART)],
            bufp, semp)

        def pack(gg, _):
            for p in range(_NBUF):
                g = gg * _NBUF + p
                pltpu.make_async_copy(
                    keys_hbm.at[pl.ds(0, 8), pl.ds(0, _CHUNK)],
                    bufs[p], sems[p]).wait()
                scan_rows(bufs[p], _CHUNK // _L, _GRP, block)

                @pl.when(g + _NBUF < _NFULL)
                def _():
                    pltpu.async_copy(
                        keys_hbm.at[pl.ds(rb, 8),
                                    pl.ds((g + _NBUF) * _CHUNK, _CHUNK)],
                        bufs[p], sems[p])

            return 0

        lax.fori_loop(0, _NFULL // _NBUF, pack, 0)

        pltpu.make_async_copy(
            keys_hbm.at[pl.ds(0, 8), pl.ds(0, _PART)], bufp, semp).wait()
        scan_rows(bufp, _PART // _L, _SUB, block)

        def fin(i, _):
            rl = block * 8 + i
            scale = row_scale(rl)
            t0 = tstate[pl.ds(i * 2 * _L, _L)]
            t1 = tstate[pl.ds(i * 2 * _L + _L, _L)]
            s0 = tv[pl.ds(rl * _K, _L)] * scale
            s1 = tv[pl.ds(rl * _K + _L, _L)] * scale
            t0, t1 = _merge_top32(t0, t1, s0)
            t0, t1 = _merge_top32(t0, t1, s1)

            d0 = _rev(t0)
            d1 = _rev(t1)
            m = t0[_L - 1]
            inv_t = jnp.float32(1.0 / _TEMP)
            e0 = jnp.exp((d0 - m) * inv_t)
            e1 = jnp.exp((d1 - m) * inv_t)
            z = _bfly(e0 + e1, jnp.add)[0]
            g0 = d0 / scale
            g1 = d1 / scale
            res0 = e0 / z * g0
            res1 = e1 / z * g1

            def qzero():
                outr[pl.ds(0, _L)] = hv[pl.ds(rl * _K, _L)] * (1.0 / 32.0)
                outr[pl.ds(_L, _L)] = hv[pl.ds(rl * _K + _L, _L)] * (1.0 / 32.0)

            def qnonzero():
                outr[pl.ds(0, _L)] = res0
                outr[pl.ds(_L, _L)] = res1

            outr[pl.ds(0, _L)] = scale
            qs = outr[pl.ds(0, _L)][0]
            lax.cond(qs == 0.0, qzero, qnonzero)

            pltpu.sync_copy(outr, out_hbm.at[pl.ds((row_base + rl) * _K, _K)])
            return 0

        lax.fori_loop(0, 8, fin, 0)


def kernel(query, keys):
    b, d = query.shape
    n = keys.shape[-1]
    k2 = keys.reshape(b * d, n)
    qf = query.reshape(b * d)
    head = keys[..., :_K].reshape(b * d * _K)
    tail = keys[..., _NMAIN:].reshape(b * d * _K)
    out_flat = _sc_topk(k2, qf, head, tail)
    return out_flat.reshape(b, d, _K).transpose(0, 2, 1)
